# baseline (speedup 1.0000x reference)
import functools

import jax
import jax.numpy as jnp
from jax import lax
from jax.experimental import pallas as pl
from jax.experimental.pallas import tpu as pltpu
from jax.experimental.pallas import tpu_sc as plsc

N = 16384
K = 50
EMB = 64
TAB = 9
KPAD = 16

NC, NS, L = 2, 16, 16
NW = NC * NS

RPB = 16
BLK = RPB * K
PER_W = N * K // NW
ROWS_W = N // NW
NBLK = ROWS_W // RPB

IB = 512

_THRESH = (2, 3, 4, 5, 8, 16, 32, 64)


def _sc_body(t_hbm, bkt_hbm, t_v0, t_v1, bkt_v, tsem0, tsem1, osem):
    t_vs = (t_v0, t_v1)
    wid = lax.axis_index("s") * NC + lax.axis_index("c")
    e_base = wid * PER_W
    i_base = wid * ROWS_W

    tsems = (tsem0, tsem1)

    def start_t(blk, u):
        pltpu.async_copy(
            t_hbm.at[pl.ds(e_base + blk * BLK, BLK)],
            t_vs[u].at[pl.ds(0, BLK)], tsems[u])

    def wait_t(u):
        pltpu.make_async_copy(
            t_hbm.at[pl.ds(0, BLK)], t_vs[u].at[pl.ds(0, BLK)],
            tsems[u]).wait()

    start_t(0, 0)
    start_t(1, 1)

    lane = lax.iota(jnp.int32, L)

    def pair_body(p, _):
        for u in (0, 1):
            blk = 2 * p + u
            wait_t(u)
            for r in range(RPB):
                i_loc = blk * RPB + r
                for k0 in range(0, K, L):
                    t = t_vs[u][pl.ds(r * K + k0, L)]
                    d = (i_base + i_loc) - t
                    b = jnp.zeros((L,), jnp.int32)
                    for thr in _THRESH:
                        b = b + jnp.where(d >= thr, 1, 0).astype(jnp.int32)
                    plsc.store_scatter(
                        bkt_v, [lane + k0, jnp.full((L,), i_loc, jnp.int32)], b)
            pl.when(blk + 2 < NBLK)(lambda: start_t(blk + 2, u))
        return 0

    lax.fori_loop(0, NBLK // 2, pair_body, 0)
    pltpu.async_copy(
        bkt_v.at[pl.ds(0, K)], bkt_hbm.at[:, 0, pl.ds(i_base, ROWS_W)], osem)
    pltpu.make_async_copy(
        bkt_v.at[pl.ds(0, K)], bkt_hbm.at[:, 0, pl.ds(0, ROWS_W)], osem).wait()


@functools.cache
def _sc_call():
    mesh = plsc.VectorSubcoreMesh(
        core_axis_name="c", subcore_axis_name="s", num_cores=NC, num_subcores=NS
    )
    return pl.kernel(
        _sc_body,
        out_type=jax.ShapeDtypeStruct((K, 1, N), jnp.int32),
        mesh=mesh,
        compiler_params=pltpu.CompilerParams(
            needs_layout_passes=False, use_tc_tiling_on_sc=False),
        scratch_types=[
            pltpu.VMEM((BLK + L,), jnp.int32),
            pltpu.VMEM((BLK + L,), jnp.int32),
            pltpu.VMEM((KPAD * 4, ROWS_W), jnp.int32),
            pltpu.SemaphoreType.DMA,
            pltpu.SemaphoreType.DMA,
            pltpu.SemaphoreType.DMA,
        ],
    )


def _tc_body(bkt_ref, tab_ref, out_ref):
    b = bkt_ref[0, 0, :]
    rows = lax.broadcasted_iota(jnp.int32, (KPAD, IB), 0)
    oh = (rows == b[None, :]).astype(jnp.float32)
    out_ref[0] = jnp.dot(tab_ref[...], oh, preferred_element_type=jnp.float32)


@functools.cache
def _tc_call():
    return pl.pallas_call(
        _tc_body,
        grid=(K, N // IB),
        in_specs=[
            pl.BlockSpec((1, 1, IB), lambda k, j: (k, 0, j)),
            pl.BlockSpec((EMB, KPAD), lambda k, j: (0, 0)),
        ],
        out_specs=pl.BlockSpec((1, EMB, IB), lambda k, j: (k, 0, j)),
        out_shape=jax.ShapeDtypeStruct((K, EMB, N), jnp.float32),
        compiler_params=pltpu.CompilerParams(
            dimension_semantics=("parallel", "parallel")),
    )


@jax.jit
def kernel(top_indices, distance_emb):
    t_flat = top_indices.reshape(-1)
    tabp = jnp.pad(distance_emb.T, ((0, 0), (0, KPAD - TAB)))
    bkt = _sc_call()(t_flat)
    out_t = _tc_call()(bkt, tabp)
    return out_t.transpose(2, 0, 1)


# speedup vs baseline: 5.7920x; 5.7920x over previous
import functools

import jax
import jax.numpy as jnp
from jax import lax
from jax.experimental import pallas as pl
from jax.experimental.pallas import tpu as pltpu
from jax.experimental.pallas import tpu_sc as plsc

N = 16384
K = 50
EMB = 64
TAB = 9
KPAD = 16

NC, NS, L = 2, 16, 16
NW = NC * NS

RPB = 16
BLK = RPB * K
PER_W = N * K // NW
ROWS_W = N // NW
NBLK = ROWS_W // RPB

IB = N

_THRESH = (2, 3, 4, 5, 8, 16, 32, 64)


def _sc_body(t_hbm, bkt_hbm, t_v0, t_v1, bkt_v, tsem0, tsem1, osem):
    t_vs = (t_v0, t_v1)
    wid = lax.axis_index("s") * NC + lax.axis_index("c")
    e_base = wid * PER_W
    i_base = wid * ROWS_W

    tsems = (tsem0, tsem1)

    def start_t(blk, u):
        pltpu.async_copy(
            t_hbm.at[pl.ds(e_base + blk * BLK, BLK)],
            t_vs[u].at[pl.ds(0, BLK)], tsems[u])

    def wait_t(u):
        pltpu.make_async_copy(
            t_hbm.at[pl.ds(0, BLK)], t_vs[u].at[pl.ds(0, BLK)],
            tsems[u]).wait()

    start_t(0, 0)
    start_t(1, 1)

    lane = lax.iota(jnp.int32, L)

    def pair_body(p, _):
        for u in (0, 1):
            blk = 2 * p + u
            wait_t(u)
            for r in range(RPB):
                i_loc = blk * RPB + r
                for k0 in range(0, K, L):
                    t = t_vs[u][pl.ds(r * K + k0, L)]
                    d = (i_base + i_loc) - t
                    b = jnp.zeros((L,), jnp.int32)
                    for thr in _THRESH:
                        b = b + jnp.where(d >= thr, 1, 0).astype(jnp.int32)
                    plsc.store_scatter(
                        bkt_v, [lane + k0, jnp.full((L,), i_loc, jnp.int32)], b)
            pl.when(blk + 2 < NBLK)(lambda: start_t(blk + 2, u))
        return 0

    lax.fori_loop(0, NBLK // 2, pair_body, 0)
    pltpu.async_copy(
        bkt_v.at[pl.ds(0, K)], bkt_hbm.at[:, 0, pl.ds(i_base, ROWS_W)], osem)
    pltpu.make_async_copy(
        bkt_v.at[pl.ds(0, K)], bkt_hbm.at[:, 0, pl.ds(0, ROWS_W)], osem).wait()


@functools.cache
def _sc_call():
    mesh = plsc.VectorSubcoreMesh(
        core_axis_name="c", subcore_axis_name="s", num_cores=NC, num_subcores=NS
    )
    return pl.kernel(
        _sc_body,
        out_type=jax.ShapeDtypeStruct((K, 1, N), jnp.int32),
        mesh=mesh,
        compiler_params=pltpu.CompilerParams(
            needs_layout_passes=False, use_tc_tiling_on_sc=False),
        scratch_types=[
            pltpu.VMEM((BLK + L,), jnp.int32),
            pltpu.VMEM((BLK + L,), jnp.int32),
            pltpu.VMEM((KPAD * 4, ROWS_W), jnp.int32),
            pltpu.SemaphoreType.DMA,
            pltpu.SemaphoreType.DMA,
            pltpu.SemaphoreType.DMA,
        ],
    )


def _tc_body(bkt_ref, tab_ref, out_ref):
    b = bkt_ref[0, 0, :]
    rows = lax.broadcasted_iota(jnp.int32, (KPAD, IB), 0)
    oh = (rows == b[None, :]).astype(jnp.float32)
    out_ref[0] = jnp.dot(tab_ref[...], oh, preferred_element_type=jnp.float32)


@functools.cache
def _tc_call():
    return pl.pallas_call(
        _tc_body,
        grid=(K,),
        in_specs=[
            pl.BlockSpec((1, 1, IB), lambda k: (k, 0, 0)),
            pl.BlockSpec((EMB, KPAD), lambda k: (0, 0)),
        ],
        out_specs=pl.BlockSpec((1, EMB, IB), lambda k: (k, 0, 0)),
        out_shape=jax.ShapeDtypeStruct((K, EMB, N), jnp.float32),
        compiler_params=pltpu.CompilerParams(
            dimension_semantics=("arbitrary",)),
    )


@jax.jit
def kernel(top_indices, distance_emb):
    t_flat = top_indices.reshape(-1)
    tabp = jnp.pad(distance_emb.T, ((0, 0), (0, KPAD - TAB)))
    bkt = _sc_call()(t_flat)
    out_t = _tc_call()(bkt, tabp)
    return out_t.transpose(2, 0, 1)
